# baseline (speedup 1.0000x reference)
import functools

import jax
import jax.numpy as jnp
from jax import lax
from jax.experimental import pallas as pl
from jax.experimental.pallas import tpu as pltpu
from jax.experimental.pallas import tpu_sc as plsc

_NC = 2
_NS = 16
_LANES = 128
_BM = 1000
_G = 64


@functools.lru_cache(maxsize=None)
def _make_sc_scatter(n, ech, q):
    nw = _NC * _NS
    rpt = -(-(n + 1) // (_NS * 8)) * 8
    np_ = rpt * _NS

    mesh = plsc.VectorSubcoreMesh(
        core_axis_name="c", subcore_axis_name="s",
        num_cores=_NC, num_subcores=_NS)

    scratch = [
        pltpu.VMEM((128,), jnp.int32),
        pltpu.VMEM((128,), jnp.int32),
        pltpu.VMEM((128,), jnp.int32),
        pltpu.VMEM((128,), jnp.int32),
        pltpu.VMEM((128, _LANES), jnp.float32),
        pltpu.VMEM((128, _LANES), jnp.float32),
        pltpu.VMEM_SHARED((np_, _LANES), jnp.float32),
        pltpu.SemaphoreType.DMA,
        pltpu.SemaphoreType.DMA,
    ]

    @functools.partial(
        pl.kernel, mesh=mesh,
        out_type=jax.ShapeDtypeStruct((q, _NC, np_, _LANES), jnp.float32),
        scratch_types=scratch,
    )
    def k(xq, srcr, dstr, zer, out, sidx0, sidx1, didx0, didx1, buf0, buf1,
          acc, sem0, sem1):
        cid = lax.axis_index("c")
        sid = lax.axis_index("s")
        wid = sid * _NC + cid
        ebase = wid * ech * 128
        r0 = sid * rpt

        for qi in range(q):
            pltpu.sync_copy(zer, acc.at[pl.ds(r0, rpt), :])
            plsc.subcore_barrier()

            def body(j, carry, qi=qi):
                base = ebase + j * 128
                pltpu.sync_copy(srcr.at[pl.ds(base, 128)], sidx0)
                pltpu.sync_copy(dstr.at[pl.ds(base, 128)], didx0)
                pltpu.async_copy(xq.at[qi].at[sidx0], buf0, sem0).wait()
                pltpu.sync_copy(buf0, acc.at[didx0], add=True)
                return carry
            lax.fori_loop(0, ech, body, 0)

            plsc.subcore_barrier()
            pltpu.sync_copy(acc.at[pl.ds(r0, rpt), :],
                            out.at[qi, cid, pl.ds(r0, rpt), :])
            plsc.subcore_barrier()

    return k


def _pad_edges(src, dst, n):
    nw = _NC * _NS
    e = src.shape[0]
    ech = -(-e // (nw * 128))
    rpt = -(-(n + 1) // (_NS * 8)) * 8
    np_ = rpt * _NS
    pad = nw * 128 * ech - e
    src_p = jnp.concatenate([src, jnp.zeros((pad,), src.dtype)])
    spare = np_ - n
    dst_pad = n + jnp.arange(pad, dtype=dst.dtype) % spare
    dst_p = jnp.concatenate([dst, dst_pad])
    return src_p, dst_p, ech


def _agg_pair(t, srcc, dstc, ech):
    n, d = t.shape
    q = d // _LANES
    xq = t.reshape(n, q, _LANES).transpose(1, 0, 2)
    rpt = -(-(n + 1) // (_NS * 8)) * 8
    zer = jnp.zeros((rpt, _LANES), jnp.float32)
    out = _make_sc_scatter(n, ech, q)(xq, srcc, dstc, zer)
    return jnp.transpose(out[:, :, :n], (1, 2, 0, 3)).reshape(_NC, n, d)


def _mm_a(x, aggp, w, b):
    n, d = x.shape
    dh = w.shape[1]

    def body(x_ref, a_ref, w_ref, b_ref, o_ref):
        h = x_ref[...] + a_ref[0] + a_ref[1]
        o_ref[...] = jnp.maximum(
            jnp.dot(h, w_ref[...], preferred_element_type=jnp.float32)
            + b_ref[...], 0.0)

    return pl.pallas_call(
        body,
        grid=(n // _BM,),
        in_specs=[
            pl.BlockSpec((_BM, d), lambda i: (i, 0)),
            pl.BlockSpec((_NC, _BM, d), lambda i: (0, i, 0)),
            pl.BlockSpec((d, dh), lambda i: (0, 0)),
            pl.BlockSpec((1, dh), lambda i: (0, 0)),
        ],
        out_specs=pl.BlockSpec((_BM, dh), lambda i: (i, 0)),
        out_shape=jax.ShapeDtypeStruct((n, dh), jnp.float32),
    )(x, aggp, w, b.reshape(1, dh))


def _mm_b(h, w, b, with_stats):
    n, d = h.shape
    dh = w.shape[1]
    nb = n // _BM

    def body(h_ref, w_ref, b_ref, y_ref, *maybe_s):
        y = (jnp.dot(h_ref[...], w_ref[...],
                     preferred_element_type=jnp.float32) + b_ref[...])
        y_ref[...] = y
        if with_stats:
            s_ref = maybe_s[0]
            z = jnp.maximum(y, 0.0)

            @pl.when(pl.program_id(0) == 0)
            def _():
                s_ref[...] = jnp.zeros_like(s_ref)
            s_ref[0:1, :] += jnp.sum(z, axis=0, keepdims=True)
            s_ref[1:2, :] += jnp.sum(z * z, axis=0, keepdims=True)

    out_shape = [jax.ShapeDtypeStruct((n, dh), jnp.float32)]
    out_specs = [pl.BlockSpec((_BM, dh), lambda i: (i, 0))]
    if with_stats:
        out_shape.append(jax.ShapeDtypeStruct((8, dh), jnp.float32))
        out_specs.append(pl.BlockSpec((8, dh), lambda i: (0, 0)))

    res = pl.pallas_call(
        body,
        grid=(nb,),
        in_specs=[
            pl.BlockSpec((_BM, d), lambda i: (i, 0)),
            pl.BlockSpec((d, dh), lambda i: (0, 0)),
            pl.BlockSpec((1, dh), lambda i: (0, 0)),
        ],
        out_specs=out_specs,
        out_shape=out_shape,
    )(h, w, b.reshape(1, dh))
    return res if with_stats else res[0]


def _bn_apply(y, s, gamma, beta):
    n, dh = y.shape
    inv_n = 1.0 / n

    def body(y_ref, s_ref, g_ref, b_ref, o_ref):
        z = jnp.maximum(y_ref[...], 0.0)
        m = s_ref[0:1, :] * inv_n
        v = s_ref[1:2, :] * inv_n - m * m
        o_ref[...] = (z - m) * lax.rsqrt(v + 1e-5) * g_ref[...] + b_ref[...]

    return pl.pallas_call(
        body,
        grid=(n // _BM,),
        in_specs=[
            pl.BlockSpec((_BM, dh), lambda i: (i, 0)),
            pl.BlockSpec((8, dh), lambda i: (0, 0)),
            pl.BlockSpec((1, dh), lambda i: (0, 0)),
            pl.BlockSpec((1, dh), lambda i: (0, 0)),
        ],
        out_specs=pl.BlockSpec((_BM, dh), lambda i: (i, 0)),
        out_shape=jax.ShapeDtypeStruct((n, dh), jnp.float32),
    )(y, s, gamma.reshape(1, dh), beta.reshape(1, dh))


def _pool(x1, x2, x3, batch2d):
    n, dh = x1.shape
    nb = n // _BM
    dtot = 3 * dh

    def body(b_ref, x1_ref, x2_ref, x3_ref, o_ref, acc, cnt):
        i = pl.program_id(0)

        @pl.when(i == 0)
        def _():
            acc[...] = jnp.zeros_like(acc)
            cnt[...] = jnp.zeros_like(cnt)

        p = (b_ref[...] == lax.broadcasted_iota(jnp.int32, (_BM, _G), 1)
             ).astype(jnp.float32)
        cnt[0:1, :] += jnp.sum(p, axis=0, keepdims=True)
        for sl in range(3):
            t_ref = (x1_ref, x2_ref, x3_ref)[sl]
            acc[:, sl * dh:(sl + 1) * dh] += lax.dot_general(
                p, t_ref[...], (((0,), (0,)), ((), ())),
                preferred_element_type=jnp.float32)

        @pl.when(i == nb - 1)
        def _():
            c = jnp.maximum(cnt[0:1, :], 1.0).reshape(_G, 1)
            o_ref[...] = acc[...] / c

    return pl.pallas_call(
        body,
        grid=(nb,),
        in_specs=[
            pl.BlockSpec((_BM, 1), lambda i: (i, 0)),
            pl.BlockSpec((_BM, dh), lambda i: (i, 0)),
            pl.BlockSpec((_BM, dh), lambda i: (i, 0)),
            pl.BlockSpec((_BM, dh), lambda i: (i, 0)),
        ],
        out_specs=pl.BlockSpec((_G, dtot), lambda i: (0, 0)),
        out_shape=jax.ShapeDtypeStruct((_G, dtot), jnp.float32),
        scratch_shapes=[
            pltpu.VMEM((_G, dtot), jnp.float32),
            pltpu.VMEM((8, _G), jnp.float32),
        ],
    )(batch2d, x1, x2, x3)


def kernel(x, edge_index, batch, W1_0, b1_0, W2_0, b2_0, W1_1, b1_1, W2_1,
           b2_1, W1_2, b1_2, W2_2, b2_2, gamma_0, beta_0, gamma_1, beta_1):
    n = x.shape[0]
    srcc, dstc, ech = _pad_edges(edge_index[0], edge_index[1], n)

    aggp = _agg_pair(x, srcc, dstc, ech)
    h = _mm_a(x, aggp, W1_0, b1_0)
    y, s = _mm_b(h, W2_0, b2_0, True)
    x1 = _bn_apply(y, s, gamma_0, beta_0)
    aggp = _agg_pair(x1, srcc, dstc, ech)
    h = _mm_a(x1, aggp, W1_1, b1_1)
    y, s = _mm_b(h, W2_1, b2_1, True)
    x2 = _bn_apply(y, s, gamma_1, beta_1)
    aggp = _agg_pair(x2, srcc, dstc, ech)
    h = _mm_a(x2, aggp, W1_2, b1_2)
    h3 = _mm_b(h, W2_2, b2_2, False)

    return _pool(x1, x2, h3, batch.reshape(n, 1))


# speedup vs baseline: 1.0006x; 1.0006x over previous
import functools

import jax
import jax.numpy as jnp
from jax import lax
from jax.experimental import pallas as pl
from jax.experimental.pallas import tpu as pltpu
from jax.experimental.pallas import tpu_sc as plsc

_NC = 2
_NS = 16
_LANES = 128
_BM = 1000
_G = 64


@functools.lru_cache(maxsize=None)
def _make_sc_scatter(n, ech, q):
    nw = _NC * _NS
    rpt = -(-(n + 1) // (_NS * 8)) * 8
    np_ = rpt * _NS

    mesh = plsc.VectorSubcoreMesh(
        core_axis_name="c", subcore_axis_name="s",
        num_cores=_NC, num_subcores=_NS)

    scratch = [
        pltpu.VMEM((128,), jnp.int32),
        pltpu.VMEM((128,), jnp.int32),
        pltpu.VMEM((128, _LANES), jnp.float32),
        pltpu.VMEM_SHARED((np_, _LANES), jnp.float32),
        pltpu.SemaphoreType.DMA,
    ]

    @functools.partial(
        pl.kernel, mesh=mesh,
        out_type=jax.ShapeDtypeStruct((q, _NC, np_, _LANES), jnp.float32),
        scratch_types=scratch,
    )
    def k(xq, srcr, dstr, zer, out, sidx0, didx0, buf0, acc, sem0):
        cid = lax.axis_index("c")
        sid = lax.axis_index("s")
        wid = sid * _NC + cid
        ebase = wid * ech * 128
        r0 = sid * rpt

        for qi in range(q):
            pltpu.sync_copy(zer, acc.at[pl.ds(r0, rpt), :])
            plsc.subcore_barrier()

            def body(j, carry, qi=qi):
                base = ebase + j * 128
                pltpu.sync_copy(srcr.at[pl.ds(base, 128)], sidx0)
                pltpu.sync_copy(dstr.at[pl.ds(base, 128)], didx0)
                pltpu.async_copy(xq.at[qi].at[sidx0], buf0, sem0).wait()
                pltpu.sync_copy(buf0, acc.at[didx0], add=True)
                return carry
            lax.fori_loop(0, ech, body, 0)

            plsc.subcore_barrier()
            pltpu.sync_copy(acc.at[pl.ds(r0, rpt), :],
                            out.at[qi, cid, pl.ds(r0, rpt), :])
            plsc.subcore_barrier()

    return k


def _pad_edges(src, dst, n):
    nw = _NC * _NS
    e = src.shape[0]
    ech = -(-e // (nw * 128))
    rpt = -(-(n + 1) // (_NS * 8)) * 8
    np_ = rpt * _NS
    pad = nw * 128 * ech - e
    src_p = jnp.concatenate([src, jnp.zeros((pad,), src.dtype)])
    spare = np_ - n
    dst_pad = n + jnp.arange(pad, dtype=dst.dtype) % spare
    dst_p = jnp.concatenate([dst, dst_pad])
    return src_p, dst_p, ech


def _agg_pair(t, srcc, dstc, ech):
    n, d = t.shape
    q = d // _LANES
    xq = t.reshape(n, q, _LANES).transpose(1, 0, 2)
    rpt = -(-(n + 1) // (_NS * 8)) * 8
    zer = jnp.zeros((rpt, _LANES), jnp.float32)
    out = _make_sc_scatter(n, ech, q)(xq, srcc, dstc, zer)
    return jnp.transpose(out[:, :, :n], (1, 2, 0, 3)).reshape(_NC, n, d)


def _mm_a(x, aggp, w, b):
    n, d = x.shape
    dh = w.shape[1]

    def body(x_ref, a_ref, w_ref, b_ref, o_ref):
        h = x_ref[...] + a_ref[0] + a_ref[1]
        o_ref[...] = jnp.maximum(
            jnp.dot(h, w_ref[...], preferred_element_type=jnp.float32)
            + b_ref[...], 0.0)

    return pl.pallas_call(
        body,
        grid=(n // _BM,),
        in_specs=[
            pl.BlockSpec((_BM, d), lambda i: (i, 0)),
            pl.BlockSpec((_NC, _BM, d), lambda i: (0, i, 0)),
            pl.BlockSpec((d, dh), lambda i: (0, 0)),
            pl.BlockSpec((1, dh), lambda i: (0, 0)),
        ],
        out_specs=pl.BlockSpec((_BM, dh), lambda i: (i, 0)),
        out_shape=jax.ShapeDtypeStruct((n, dh), jnp.float32),
    )(x, aggp, w, b.reshape(1, dh))


def _mm_b(h, w, b, with_stats):
    n, d = h.shape
    dh = w.shape[1]
    nb = n // _BM

    def body(h_ref, w_ref, b_ref, y_ref, *maybe_s):
        y = (jnp.dot(h_ref[...], w_ref[...],
                     preferred_element_type=jnp.float32) + b_ref[...])
        y_ref[...] = y
        if with_stats:
            s_ref = maybe_s[0]
            z = jnp.maximum(y, 0.0)

            @pl.when(pl.program_id(0) == 0)
            def _():
                s_ref[...] = jnp.zeros_like(s_ref)
            s_ref[0:1, :] += jnp.sum(z, axis=0, keepdims=True)
            s_ref[1:2, :] += jnp.sum(z * z, axis=0, keepdims=True)

    out_shape = [jax.ShapeDtypeStruct((n, dh), jnp.float32)]
    out_specs = [pl.BlockSpec((_BM, dh), lambda i: (i, 0))]
    if with_stats:
        out_shape.append(jax.ShapeDtypeStruct((8, dh), jnp.float32))
        out_specs.append(pl.BlockSpec((8, dh), lambda i: (0, 0)))

    res = pl.pallas_call(
        body,
        grid=(nb,),
        in_specs=[
            pl.BlockSpec((_BM, d), lambda i: (i, 0)),
            pl.BlockSpec((d, dh), lambda i: (0, 0)),
            pl.BlockSpec((1, dh), lambda i: (0, 0)),
        ],
        out_specs=out_specs,
        out_shape=out_shape,
    )(h, w, b.reshape(1, dh))
    return res if with_stats else res[0]


def _bn_apply(y, s, gamma, beta):
    n, dh = y.shape
    inv_n = 1.0 / n

    def body(y_ref, s_ref, g_ref, b_ref, o_ref):
        z = jnp.maximum(y_ref[...], 0.0)
        m = s_ref[0:1, :] * inv_n
        v = s_ref[1:2, :] * inv_n - m * m
        o_ref[...] = (z - m) * lax.rsqrt(v + 1e-5) * g_ref[...] + b_ref[...]

    return pl.pallas_call(
        body,
        grid=(n // _BM,),
        in_specs=[
            pl.BlockSpec((_BM, dh), lambda i: (i, 0)),
            pl.BlockSpec((8, dh), lambda i: (0, 0)),
            pl.BlockSpec((1, dh), lambda i: (0, 0)),
            pl.BlockSpec((1, dh), lambda i: (0, 0)),
        ],
        out_specs=pl.BlockSpec((_BM, dh), lambda i: (i, 0)),
        out_shape=jax.ShapeDtypeStruct((n, dh), jnp.float32),
    )(y, s, gamma.reshape(1, dh), beta.reshape(1, dh))


def _pool(x1, x2, x3, batch2d):
    n, dh = x1.shape
    nb = n // _BM
    dtot = 3 * dh

    def body(b_ref, x1_ref, x2_ref, x3_ref, o_ref, acc, cnt):
        i = pl.program_id(0)

        @pl.when(i == 0)
        def _():
            acc[...] = jnp.zeros_like(acc)
            cnt[...] = jnp.zeros_like(cnt)

        p = (b_ref[...] == lax.broadcasted_iota(jnp.int32, (_BM, _G), 1)
             ).astype(jnp.float32)
        cnt[0:1, :] += jnp.sum(p, axis=0, keepdims=True)
        for sl in range(3):
            t_ref = (x1_ref, x2_ref, x3_ref)[sl]
            acc[:, sl * dh:(sl + 1) * dh] += lax.dot_general(
                p, t_ref[...], (((0,), (0,)), ((), ())),
                preferred_element_type=jnp.float32)

        @pl.when(i == nb - 1)
        def _():
            c = jnp.maximum(cnt[0:1, :], 1.0).reshape(_G, 1)
            o_ref[...] = acc[...] / c

    return pl.pallas_call(
        body,
        grid=(nb,),
        in_specs=[
            pl.BlockSpec((_BM, 1), lambda i: (i, 0)),
            pl.BlockSpec((_BM, dh), lambda i: (i, 0)),
            pl.BlockSpec((_BM, dh), lambda i: (i, 0)),
            pl.BlockSpec((_BM, dh), lambda i: (i, 0)),
        ],
        out_specs=pl.BlockSpec((_G, dtot), lambda i: (0, 0)),
        out_shape=jax.ShapeDtypeStruct((_G, dtot), jnp.float32),
        scratch_shapes=[
            pltpu.VMEM((_G, dtot), jnp.float32),
            pltpu.VMEM((8, _G), jnp.float32),
        ],
    )(batch2d, x1, x2, x3)


def kernel(x, edge_index, batch, W1_0, b1_0, W2_0, b2_0, W1_1, b1_1, W2_1,
           b2_1, W1_2, b1_2, W2_2, b2_2, gamma_0, beta_0, gamma_1, beta_1):
    n = x.shape[0]
    srcc, dstc, ech = _pad_edges(edge_index[0], edge_index[1], n)

    aggp = _agg_pair(x, srcc, dstc, ech)
    h = _mm_a(x, aggp, W1_0, b1_0)
    y, s = _mm_b(h, W2_0, b2_0, True)
    x1 = _bn_apply(y, s, gamma_0, beta_0)
    aggp = _agg_pair(x1, srcc, dstc, ech)
    h = _mm_a(x1, aggp, W1_1, b1_1)
    y, s = _mm_b(h, W2_1, b2_1, True)
    x2 = _bn_apply(y, s, gamma_1, beta_1)
    aggp = _agg_pair(x2, srcc, dstc, ech)
    h = _mm_a(x2, aggp, W1_2, b1_2)
    h3 = _mm_b(h, W2_2, b2_2, False)

    return _pool(x1, x2, h3, batch.reshape(n, 1))


# speedup vs baseline: 2.0028x; 2.0016x over previous
import functools

import jax
import jax.numpy as jnp
from jax import lax
from jax.experimental import pallas as pl
from jax.experimental.pallas import tpu as pltpu
from jax.experimental.pallas import tpu_sc as plsc

_NC = 2
_NS = 16
_LANES = 128
_BM = 1000
_G = 64


@functools.lru_cache(maxsize=None)
def _make_sc_scatter(n, e, q):
    nw = _NC * _NS
    ept = e // nw
    assert ept * nw == e
    full = ept // 128
    tail = ept - full * 128
    assert tail == 0 or tail % 8 == 0
    tb = max(tail, 8)
    rpt = -(-(n + 1) // (_NS * 8)) * 8
    np_ = rpt * _NS

    mesh = plsc.VectorSubcoreMesh(
        core_axis_name="c", subcore_axis_name="s",
        num_cores=_NC, num_subcores=_NS)

    scratch = [
        pltpu.VMEM((128,), jnp.int32),
        pltpu.VMEM((128,), jnp.int32),
        pltpu.VMEM((128, _LANES), jnp.float32),
        pltpu.VMEM((tb,), jnp.int32),
        pltpu.VMEM((tb,), jnp.int32),
        pltpu.VMEM((tb, _LANES), jnp.float32),
        pltpu.VMEM_SHARED((np_, _LANES), jnp.float32),
        pltpu.SemaphoreType.DMA,
    ]

    @functools.partial(
        pl.kernel, mesh=mesh,
        out_type=jax.ShapeDtypeStruct((q, _NC, np_, _LANES), jnp.float32),
        scratch_types=scratch,
    )
    def k(xq, srcr, dstr, zer, out, sidx0, didx0, buf0, sidxt, didxt, rowst,
          acc, sem0):
        cid = lax.axis_index("c")
        sid = lax.axis_index("s")
        wid = sid * _NC + cid
        ebase = wid * ept
        r0 = sid * rpt

        def chunk(off, sbuf, dbuf, rbuf, qi, ce):
            pltpu.sync_copy(srcr.at[pl.ds(off, ce)], sbuf)
            pltpu.sync_copy(dstr.at[pl.ds(off, ce)], dbuf)
            pltpu.async_copy(xq.at[qi].at[sbuf], rbuf, sem0).wait()
            pltpu.sync_copy(rbuf, acc.at[dbuf], add=True)

        for qi in range(q):
            pltpu.sync_copy(zer, acc.at[pl.ds(r0, rpt), :])
            plsc.subcore_barrier()

            def body(j, carry, qi=qi):
                chunk(ebase + j * 128, sidx0, didx0, buf0, qi, 128)
                return carry
            if full:
                lax.fori_loop(0, full, body, 0)
            if tail:
                chunk(ebase + full * 128, sidxt, didxt, rowst, qi, tail)

            plsc.subcore_barrier()
            pltpu.sync_copy(acc.at[pl.ds(r0, rpt), :],
                            out.at[qi, cid, pl.ds(r0, rpt), :])
            plsc.subcore_barrier()

    return k


def _pad_edges(src, dst, n):
    return src, dst, src.shape[0]


def _agg_pair(t, srcc, dstc, ech):
    n, d = t.shape
    q = d // _LANES
    xq = t.reshape(n, q, _LANES).transpose(1, 0, 2)
    rpt = -(-(n + 1) // (_NS * 8)) * 8
    zer = jnp.zeros((rpt, _LANES), jnp.float32)
    out = _make_sc_scatter(n, ech, q)(xq, srcc, dstc, zer)
    return jnp.transpose(out[:, :, :n], (1, 2, 0, 3)).reshape(_NC, n, d)


def _mm_a(x, aggp, w, b):
    n, d = x.shape
    dh = w.shape[1]

    def body(x_ref, a_ref, w_ref, b_ref, o_ref):
        h = x_ref[...] + a_ref[0] + a_ref[1]
        o_ref[...] = jnp.maximum(
            jnp.dot(h, w_ref[...], preferred_element_type=jnp.float32)
            + b_ref[...], 0.0)

    return pl.pallas_call(
        body,
        grid=(n // _BM,),
        in_specs=[
            pl.BlockSpec((_BM, d), lambda i: (i, 0)),
            pl.BlockSpec((_NC, _BM, d), lambda i: (0, i, 0)),
            pl.BlockSpec((d, dh), lambda i: (0, 0)),
            pl.BlockSpec((1, dh), lambda i: (0, 0)),
        ],
        out_specs=pl.BlockSpec((_BM, dh), lambda i: (i, 0)),
        out_shape=jax.ShapeDtypeStruct((n, dh), jnp.float32),
    )(x, aggp, w, b.reshape(1, dh))


def _mm_b(h, w, b, with_stats):
    n, d = h.shape
    dh = w.shape[1]
    nb = n // _BM

    def body(h_ref, w_ref, b_ref, y_ref, *maybe_s):
        y = (jnp.dot(h_ref[...], w_ref[...],
                     preferred_element_type=jnp.float32) + b_ref[...])
        y_ref[...] = y
        if with_stats:
            s_ref = maybe_s[0]
            z = jnp.maximum(y, 0.0)

            @pl.when(pl.program_id(0) == 0)
            def _():
                s_ref[...] = jnp.zeros_like(s_ref)
            s_ref[0:1, :] += jnp.sum(z, axis=0, keepdims=True)
            s_ref[1:2, :] += jnp.sum(z * z, axis=0, keepdims=True)

    out_shape = [jax.ShapeDtypeStruct((n, dh), jnp.float32)]
    out_specs = [pl.BlockSpec((_BM, dh), lambda i: (i, 0))]
    if with_stats:
        out_shape.append(jax.ShapeDtypeStruct((8, dh), jnp.float32))
        out_specs.append(pl.BlockSpec((8, dh), lambda i: (0, 0)))

    res = pl.pallas_call(
        body,
        grid=(nb,),
        in_specs=[
            pl.BlockSpec((_BM, d), lambda i: (i, 0)),
            pl.BlockSpec((d, dh), lambda i: (0, 0)),
            pl.BlockSpec((1, dh), lambda i: (0, 0)),
        ],
        out_specs=out_specs,
        out_shape=out_shape,
    )(h, w, b.reshape(1, dh))
    return res if with_stats else res[0]


def _bn_apply(y, s, gamma, beta):
    n, dh = y.shape
    inv_n = 1.0 / n

    def body(y_ref, s_ref, g_ref, b_ref, o_ref):
        z = jnp.maximum(y_ref[...], 0.0)
        m = s_ref[0:1, :] * inv_n
        v = s_ref[1:2, :] * inv_n - m * m
        o_ref[...] = (z - m) * lax.rsqrt(v + 1e-5) * g_ref[...] + b_ref[...]

    return pl.pallas_call(
        body,
        grid=(n // _BM,),
        in_specs=[
            pl.BlockSpec((_BM, dh), lambda i: (i, 0)),
            pl.BlockSpec((8, dh), lambda i: (0, 0)),
            pl.BlockSpec((1, dh), lambda i: (0, 0)),
            pl.BlockSpec((1, dh), lambda i: (0, 0)),
        ],
        out_specs=pl.BlockSpec((_BM, dh), lambda i: (i, 0)),
        out_shape=jax.ShapeDtypeStruct((n, dh), jnp.float32),
    )(y, s, gamma.reshape(1, dh), beta.reshape(1, dh))


def _pool(x1, x2, x3, batch2d):
    n, dh = x1.shape
    nb = n // _BM
    dtot = 3 * dh

    def body(b_ref, x1_ref, x2_ref, x3_ref, o_ref, acc, cnt):
        i = pl.program_id(0)

        @pl.when(i == 0)
        def _():
            acc[...] = jnp.zeros_like(acc)
            cnt[...] = jnp.zeros_like(cnt)

        p = (b_ref[...] == lax.broadcasted_iota(jnp.int32, (_BM, _G), 1)
             ).astype(jnp.float32)
        cnt[0:1, :] += jnp.sum(p, axis=0, keepdims=True)
        for sl in range(3):
            t_ref = (x1_ref, x2_ref, x3_ref)[sl]
            acc[:, sl * dh:(sl + 1) * dh] += lax.dot_general(
                p, t_ref[...], (((0,), (0,)), ((), ())),
                preferred_element_type=jnp.float32)

        @pl.when(i == nb - 1)
        def _():
            c = jnp.maximum(cnt[0:1, :], 1.0).reshape(_G, 1)
            o_ref[...] = acc[...] / c

    return pl.pallas_call(
        body,
        grid=(nb,),
        in_specs=[
            pl.BlockSpec((_BM, 1), lambda i: (i, 0)),
            pl.BlockSpec((_BM, dh), lambda i: (i, 0)),
            pl.BlockSpec((_BM, dh), lambda i: (i, 0)),
            pl.BlockSpec((_BM, dh), lambda i: (i, 0)),
        ],
        out_specs=pl.BlockSpec((_G, dtot), lambda i: (0, 0)),
        out_shape=jax.ShapeDtypeStruct((_G, dtot), jnp.float32),
        scratch_shapes=[
            pltpu.VMEM((_G, dtot), jnp.float32),
            pltpu.VMEM((8, _G), jnp.float32),
        ],
    )(batch2d, x1, x2, x3)


def kernel(x, edge_index, batch, W1_0, b1_0, W2_0, b2_0, W1_1, b1_1, W2_1,
           b2_1, W1_2, b1_2, W2_2, b2_2, gamma_0, beta_0, gamma_1, beta_1):
    n = x.shape[0]
    srcc, dstc, ech = _pad_edges(edge_index[0], edge_index[1], n)

    aggp = _agg_pair(x, srcc, dstc, ech)
    h = _mm_a(x, aggp, W1_0, b1_0)
    y, s = _mm_b(h, W2_0, b2_0, True)
    x1 = _bn_apply(y, s, gamma_0, beta_0)
    aggp = _agg_pair(x1, srcc, dstc, ech)
    h = _mm_a(x1, aggp, W1_1, b1_1)
    y, s = _mm_b(h, W2_1, b2_1, True)
    x2 = _bn_apply(y, s, gamma_1, beta_1)
    aggp = _agg_pair(x2, srcc, dstc, ech)
    h = _mm_a(x2, aggp, W1_2, b1_2)
    h3 = _mm_b(h, W2_2, b2_2, False)

    return _pool(x1, x2, h3, batch.reshape(n, 1))
